# baseline (speedup 1.0000x reference)
import functools

import jax
import jax.numpy as jnp
from jax import lax
from jax.experimental import pallas as pl
from jax.experimental.pallas import tpu as pltpu
from jax.experimental.pallas import tpu_sc as plsc

N = 10000
E = 320000
C = 128
H = 64
NC = 2
NS = 16
K = 128
NCH = 160
E_PAD = NS * NCH * K
NROW = E_PAD // K
NCW = NROW // (NC * NS)
NPAD = 64
AR = N + 248
ARD = N + 128
ZRA = 632
ZR = 624
ZR_LAST = N - (NS - 1) * ZR

_mesh = plsc.VectorSubcoreMesh(core_axis_name="c", subcore_axis_name="s")


@functools.partial(
    pl.kernel,
    out_type=jax.ShapeDtypeStruct((NC, 1, ARD), jnp.float32),
    mesh=_mesh,
    scratch_types=[
        pltpu.VMEM((NCW, K), jnp.int32),
        pltpu.VMEM((K,), jnp.float32),
        pltpu.VMEM((ARD,), jnp.float32),
        pltpu.VMEM_SHARED((ARD,), jnp.float32),
        pltpu.SemaphoreType.DMA,
    ],
)
def _deg_kernel(dst3_hbm, deg_out, idx_v, ones_v, zbuf, acc, sem):
    cid = lax.axis_index("c")
    sid = lax.axis_index("s")
    wid = cid * NS + sid

    cp = pltpu.async_copy(dst3_hbm.at[pl.ds(wid * NCW, NCW)], idx_v, sem)

    for j in range(K // 16):
        ones_v[pl.ds(j * 16, 16)] = jnp.ones((16,), jnp.float32)

    @pl.when(sid == 0)
    def _zero():
        def zrow(i, _):
            zbuf[pl.ds(i * 16, 16)] = jnp.zeros((16,), jnp.float32)
            return 0
        lax.fori_loop(0, ARD // 16, zrow, 0)
        pltpu.sync_copy(zbuf, acc)

    cp.wait()
    plsc.subcore_barrier()

    def chunk(i, _):
        pltpu.sync_copy(ones_v, acc.at[idx_v.at[i]], add=True)
        return 0

    lax.fori_loop(0, NCW, chunk, 0)
    plsc.subcore_barrier()

    @pl.when(sid == 0)
    def _out():
        pltpu.sync_copy(acc, deg_out.at[cid, 0])


@functools.partial(
    pl.kernel,
    out_type=jax.ShapeDtypeStruct((NC, N, H), jnp.float32),
    mesh=_mesh,
    scratch_types=[
        pltpu.VMEM((NCH, K), jnp.int32),
        pltpu.VMEM((NCH, K), jnp.int32),
        pltpu.VMEM((K, H), jnp.float32),
        pltpu.VMEM((K, H), jnp.float32),
        pltpu.VMEM((K, H), jnp.float32),
        pltpu.VMEM((K, H), jnp.float32),
        pltpu.VMEM_SHARED((AR, H), jnp.float32),
        pltpu.SemaphoreType.DMA,
        pltpu.SemaphoreType.DMA,
        pltpu.SemaphoreType.DMA,
        pltpu.SemaphoreType.DMA,
        pltpu.SemaphoreType.DMA,
        pltpu.SemaphoreType.DMA,
        pltpu.SemaphoreType.DMA,
        pltpu.SemaphoreType.DMA,
        pltpu.SemaphoreType.DMA,
    ],
    compiler_params=pltpu.CompilerParams(use_tc_tiling_on_sc=False),
)
def _prop_kernel(u_hbm, src3_hbm, dst3_hbm, out_hbm, sidx, didx,
                 r0, r1, r2, r3, acc, semi,
                 g0, g1, g2, g3, s0, s1, s2, s3):
    cid = lax.axis_index("c")
    sid = lax.axis_index("s")
    rows = (r0, r1, r2, r3)
    gsem = (g0, g1, g2, g3)
    ssem = (s0, s1, s2, s3)

    cps = pltpu.async_copy(src3_hbm.at[cid, pl.ds(sid * NCH, NCH)], sidx, semi)
    cpd = pltpu.async_copy(dst3_hbm.at[pl.ds(sid * NCH, NCH)], didx, semi)

    def zrow(i, _):
        for j in range(H // 16):
            r0[i, pl.ds(j * 16, 16)] = jnp.zeros((16,), jnp.float32)
        return 0

    lax.fori_loop(0, K, zrow, 0)
    zb = sid * ZRA
    for t in range(5):
        pltpu.sync_copy(r0, acc.at[pl.ds(zb + t * K, K)])

    @pl.when(sid == NS - 1)
    def _z_tail():
        pltpu.sync_copy(r0, acc.at[pl.ds(zb + 5 * K, K)])
    cps.wait()
    cpd.wait()
    plsc.subcore_barrier()

    for b in range(3):
        pltpu.async_copy(u_hbm.at[sidx.at[b]], rows[b], gsem[b])

    def quad(q, _):
        for t in range(4):
            i = q * 4 + t
            b = t
            bp = (t + 3) % 4
            bn = (t + 3) % 4
            pltpu.make_async_copy(u_hbm.at[sidx.at[i]], rows[b],
                                  gsem[b]).wait()
            pltpu.async_copy(rows[b], acc.at[didx.at[i]], ssem[b], add=True)

            @pl.when(i >= 1)
            def _wait_prev_scatter():
                pltpu.make_async_copy(rows[bp], acc.at[didx.at[i - 1]],
                                      ssem[bp]).wait()

            @pl.when(i + 3 < NCH)
            def _prefetch():
                pltpu.async_copy(u_hbm.at[sidx.at[i + 3]], rows[bn], gsem[bn])
        return 0

    lax.fori_loop(0, NCH // 4, quad, 0)
    pltpu.make_async_copy(r3, acc.at[didx.at[NCH - 1]], ssem[3]).wait()
    plsc.subcore_barrier()

    @pl.when(sid < NS - 1)
    def _out_body():
        pltpu.sync_copy(acc.at[pl.ds(sid * ZR, ZR)],
                        out_hbm.at[cid, pl.ds(sid * ZR, ZR)])

    @pl.when(sid == NS - 1)
    def _out_last():
        pltpu.sync_copy(acc.at[pl.ds((NS - 1) * ZR, ZR_LAST)],
                        out_hbm.at[cid, pl.ds((NS - 1) * ZR, ZR_LAST)])


BR = 1000
NB = N // BR


def _dense1_body(x_ref, w_ref, degs_ref, u_ref):
    dinv = lax.rsqrt(degs_ref[...])
    xw = jnp.dot(x_ref[...], w_ref[0], preferred_element_type=jnp.float32)
    u_ref[...] = xw * dinv


def _dense2_body(s_ref, u_ref, degs_ref, b_ref, u2_ref):
    dinv = lax.rsqrt(degs_ref[...])
    pre = dinv * (s_ref[0] + u_ref[...]) + b_ref[0]
    u2_ref[...] = dinv * jnp.maximum(pre, 0.0)


def _dense3_body(s_ref, ulo_ref, uhi_ref, degs_ref, wmu_ref, bmu_ref,
                 wls_ref, bls_ref, mu_ref, ls_ref):
    dinv = lax.rsqrt(degs_ref[...])
    glo = dinv * (s_ref[0] + ulo_ref[...])
    ghi = dinv * (s_ref[1] + uhi_ref[...])
    mu_ref[...] = (jnp.dot(glo, wmu_ref[0:H], preferred_element_type=jnp.float32)
                   + jnp.dot(ghi, wmu_ref[H:C], preferred_element_type=jnp.float32)
                   + bmu_ref[...])
    ls_ref[...] = (jnp.dot(glo, wls_ref[0:H], preferred_element_type=jnp.float32)
                   + jnp.dot(ghi, wls_ref[H:C], preferred_element_type=jnp.float32)
                   + bls_ref[...])


def kernel(x, edge_index, W1, b1, Wmu, bmu, Wls, bls):
    ei = edge_index.astype(jnp.int32)
    src, dst = ei[0], ei[1]

    pad = jnp.arange(E_PAD - E, dtype=jnp.int32) % NPAD
    src_p = jnp.concatenate([src, pad])
    dst_p = jnp.concatenate([dst, N + pad])
    dst3 = dst_p.reshape(NROW, K)
    src3 = jnp.stack([src_p, src_p + N]).reshape(NC, NROW, K)

    deg2 = _deg_kernel(dst3)
    degs = (deg2[0, 0, :N] + deg2[1, 0, :N] + 1.0).reshape(N, 1)

    u1 = pl.pallas_call(
        _dense1_body,
        grid=(NC, NB),
        in_specs=[
            pl.BlockSpec((BR, C), lambda c, i: (i, 0)),
            pl.BlockSpec((1, C, H), lambda c, i: (c, 0, 0)),
            pl.BlockSpec((BR, 1), lambda c, i: (i, 0)),
        ],
        out_specs=pl.BlockSpec((BR, H), lambda c, i: (c * NB + i, 0)),
        out_shape=jax.ShapeDtypeStruct((NC * N, H), jnp.float32),
    )(x, W1.reshape(C, NC, H).transpose(1, 0, 2), degs)

    s1 = _prop_kernel(u1, src3, dst3)

    u2 = pl.pallas_call(
        _dense2_body,
        grid=(NC, NB),
        in_specs=[
            pl.BlockSpec((1, BR, H), lambda c, i: (c, i, 0)),
            pl.BlockSpec((BR, H), lambda c, i: (c * NB + i, 0)),
            pl.BlockSpec((BR, 1), lambda c, i: (i, 0)),
            pl.BlockSpec((1, 1, H), lambda c, i: (c, 0, 0)),
        ],
        out_specs=pl.BlockSpec((BR, H), lambda c, i: (c * NB + i, 0)),
        out_shape=jax.ShapeDtypeStruct((NC * N, H), jnp.float32),
    )(s1, u1, degs, b1.reshape(NC, 1, H))

    s2 = _prop_kernel(u2, src3, dst3)

    OC = Wmu.shape[1]
    mu, ls = pl.pallas_call(
        _dense3_body,
        grid=(NB,),
        in_specs=[
            pl.BlockSpec((2, BR, H), lambda i: (0, i, 0)),
            pl.BlockSpec((BR, H), lambda i: (i, 0)),
            pl.BlockSpec((BR, H), lambda i: (NB + i, 0)),
            pl.BlockSpec((BR, 1), lambda i: (i, 0)),
            pl.BlockSpec((C, OC), lambda i: (0, 0)),
            pl.BlockSpec((1, OC), lambda i: (0, 0)),
            pl.BlockSpec((C, OC), lambda i: (0, 0)),
            pl.BlockSpec((1, OC), lambda i: (0, 0)),
        ],
        out_specs=[pl.BlockSpec((BR, OC), lambda i: (i, 0)),
                   pl.BlockSpec((BR, OC), lambda i: (i, 0))],
        out_shape=[jax.ShapeDtypeStruct((N, OC), jnp.float32),
                   jax.ShapeDtypeStruct((N, OC), jnp.float32)],
    )(s2, u2, u2, degs, Wmu, bmu.reshape(1, OC), Wls, bls.reshape(1, OC))

    return (mu, ls)
